# baseline (speedup 1.0000x reference)
import functools

import jax
import jax.numpy as jnp
from jax import lax
from jax.experimental import pallas as pl
from jax.experimental.pallas import tpu as pltpu
from jax.experimental.pallas import tpu_sc as plsc

N_E = 8192
E_DIM = 32
N_ROWS = 8192
ROW_BLK = 256
CODE_BLK = 1024
N_ROW_BLKS = N_ROWS // ROW_BLK
N_CODE_BLKS = N_E // CODE_BLK

_MATMUL_PRECISION = lax.Precision.DEFAULT


def _argmin_body(zt_ref, cb_ref, idx_ref, loss_ref, cbt_ref, esq_ref):
    i = pl.program_id(0)
    zt = zt_ref[0]

    @pl.when(i == 0)
    def _():
        cbt = jnp.transpose(cb_ref[...])
        cbt_ref[...] = cbt
        esq_ref[...] = jnp.sum(cbt * cbt, axis=0, keepdims=True)

    zsq = jnp.sum(zt * zt, axis=0)[:, None]
    z2 = -2.0 * zt
    colf = lax.broadcasted_iota(
        jnp.int32, (ROW_BLK, CODE_BLK), 1).astype(jnp.float32)
    best_val = jnp.full((ROW_BLK, 1), jnp.inf, dtype=jnp.float32)
    best_col = jnp.zeros((ROW_BLK, 1), dtype=jnp.float32)
    for c in range(N_CODE_BLKS):
        sl = slice(c * CODE_BLK, (c + 1) * CODE_BLK)
        cbt = cbt_ref[:, sl]
        esq = esq_ref[:, sl]
        m = lax.dot_general(
            z2, cbt, (((0,), (0,)), ((), ())),
            precision=_MATMUL_PRECISION,
            preferred_element_type=jnp.float32)
        d = (zsq + esq) + m
        cmin = jnp.min(d, axis=1, keepdims=True)
        ccol = jnp.min(jnp.where(d == cmin, colf, float(N_E)),
                       axis=1, keepdims=True) + float(c * CODE_BLK)
        take = cmin < best_val
        best_val = jnp.where(take, cmin, best_val)
        best_col = jnp.where(take, ccol, best_col)
    idx_ref[0, 0, :] = best_col[:, 0].astype(jnp.int32)
    part = jnp.sum(best_val)

    @pl.when(i == 0)
    def _():
        loss_ref[0, 0] = part

    @pl.when(i != 0)
    def _():
        loss_ref[0, 0] += part


def _argmin_call(z3, codebook):
    blks_per_batch = 1024 // ROW_BLK
    return pl.pallas_call(
        _argmin_body,
        grid=(N_ROW_BLKS,),
        in_specs=[
            pl.BlockSpec((1, E_DIM, ROW_BLK),
                         lambda i: (i // blks_per_batch, 0, i % blks_per_batch)),
            pl.BlockSpec((N_E, E_DIM), lambda i: (0, 0)),
        ],
        out_specs=[
            pl.BlockSpec((1, 1, ROW_BLK), lambda i: (i, 0, 0)),
            pl.BlockSpec(memory_space=pltpu.SMEM, block_shape=(1, 1),
                         index_map=lambda i: (0, 0)),
        ],
        out_shape=[
            jax.ShapeDtypeStruct((N_ROW_BLKS, 1, ROW_BLK), jnp.int32),
            jax.ShapeDtypeStruct((1, 1), jnp.float32),
        ],
        scratch_shapes=[
            pltpu.VMEM((E_DIM, N_E), jnp.float32),
            pltpu.VMEM((1, N_E), jnp.float32),
        ],
    )(z3, codebook)


_SC_CHUNK = 128


def _make_gather():
    info = plsc.get_sparse_core_info()
    nw = info.num_cores * info.num_subcores
    per_w = N_ROWS // nw
    n_chunks = per_w // _SC_CHUNK
    mesh = plsc.VectorSubcoreMesh(core_axis_name="c", subcore_axis_name="s")

    @functools.partial(
        pl.kernel,
        mesh=mesh,
        compiler_params=pltpu.CompilerParams(use_tc_tiling_on_sc=False),
        out_type=jax.ShapeDtypeStruct((N_ROWS, E_DIM), jnp.float32),
        scratch_types=[
            pltpu.VMEM((_SC_CHUNK,), jnp.int32),
            pltpu.VMEM((_SC_CHUNK, E_DIM), jnp.float32),
            pltpu.SemaphoreType.DMA,
        ],
    )
    def gather(table_hbm, idx_hbm, out_hbm, idx_v, rows_v, sem):
        wid = lax.axis_index("s") * info.num_cores + lax.axis_index("c")
        base = wid * per_w
        for k in range(n_chunks):
            off = base + k * _SC_CHUNK
            pltpu.sync_copy(idx_hbm.at[pl.ds(off, _SC_CHUNK)], idx_v)
            pltpu.async_copy(table_hbm.at[idx_v], rows_v, sem).wait()
            pltpu.sync_copy(rows_v, out_hbm.at[pl.ds(off, _SC_CHUNK)])

    return gather


def kernel(z, codebook):
    z3 = z.reshape(z.shape[0], E_DIM, -1)

    idx_blocks, loss_sum = _argmin_call(z3, codebook)
    idx = idx_blocks.reshape(N_ROWS)

    z_q_flat = _make_gather()(codebook, idx)

    z_q = z_q_flat.reshape(z.shape[0], z.shape[2], z.shape[3], E_DIM)
    z_q_out = jnp.transpose(z_q, (0, 3, 1, 2))
    codebook_loss = loss_sum[0, 0] / (N_ROWS * E_DIM)
    idx_map = idx.reshape(z.shape[0], 1, z.shape[2], z.shape[3])
    return (z_q_out, codebook_loss, 0, idx_map)


# speedup vs baseline: 1.1724x; 1.1724x over previous
import functools

import jax
import jax.numpy as jnp
from jax import lax
from jax.experimental import pallas as pl
from jax.experimental.pallas import tpu as pltpu
from jax.experimental.pallas import tpu_sc as plsc

N_E = 8192
E_DIM = 32
N_ROWS = 8192
ROW_BLK = 256
CODE_BLK = 1024
N_ROW_BLKS = N_ROWS // ROW_BLK
N_CODE_BLKS = N_E // CODE_BLK

_MATMUL_PRECISION = lax.Precision.DEFAULT


def _argmin_body(zt_ref, cbt_ref, idx_ref, loss_ref, esq_ref):
    i = pl.program_id(0)
    zt = zt_ref[0]

    @pl.when(i == 0)
    def _():
        cb = cbt_ref[...]
        esq_ref[...] = jnp.sum(cb * cb, axis=0, keepdims=True)

    zsq = jnp.sum(zt * zt, axis=0)[:, None]
    z2 = -2.0 * zt
    LANES = 128
    run_val = jnp.full((ROW_BLK, LANES), jnp.inf, dtype=jnp.float32)
    run_k = jnp.zeros((ROW_BLK, LANES), dtype=jnp.float32)
    for c in range(N_CODE_BLKS):
        sl = slice(c * CODE_BLK, (c + 1) * CODE_BLK)
        cbt = cbt_ref[:, sl]
        m = lax.dot_general(
            z2, cbt, (((0,), (0,)), ((), ())),
            precision=_MATMUL_PRECISION,
            preferred_element_type=jnp.float32)
        for s in range(CODE_BLK // LANES):
            k = c * (CODE_BLK // LANES) + s
            esq = esq_ref[:, k * LANES:(k + 1) * LANES]
            d = (zsq + esq) + m[:, s * LANES:(s + 1) * LANES]
            mask = d < run_val
            run_val = jnp.where(mask, d, run_val)
            run_k = jnp.where(mask, jnp.float32(k), run_k)
    gmin = jnp.min(run_val, axis=1, keepdims=True)
    lane = lax.broadcasted_iota(
        jnp.int32, (ROW_BLK, LANES), 1).astype(jnp.float32)
    cand = jnp.where(run_val == gmin, run_k * float(LANES) + lane, 1e9)
    best_col = jnp.min(cand, axis=1)
    idx_ref[0, 0, :] = best_col.astype(jnp.int32)
    part = jnp.sum(gmin)

    @pl.when(i == 0)
    def _():
        loss_ref[0, 0] = part

    @pl.when(i != 0)
    def _():
        loss_ref[0, 0] += part


def _argmin_call(z3, cbt):
    blks_per_batch = 1024 // ROW_BLK
    return pl.pallas_call(
        _argmin_body,
        grid=(N_ROW_BLKS,),
        in_specs=[
            pl.BlockSpec((1, E_DIM, ROW_BLK),
                         lambda i: (i // blks_per_batch, 0, i % blks_per_batch)),
            pl.BlockSpec((E_DIM, N_E), lambda i: (0, 0)),
        ],
        out_specs=[
            pl.BlockSpec((1, 1, ROW_BLK), lambda i: (i, 0, 0)),
            pl.BlockSpec(memory_space=pltpu.SMEM, block_shape=(1, 1),
                         index_map=lambda i: (0, 0)),
        ],
        out_shape=[
            jax.ShapeDtypeStruct((N_ROW_BLKS, 1, ROW_BLK), jnp.int32),
            jax.ShapeDtypeStruct((1, 1), jnp.float32),
        ],
        scratch_shapes=[pltpu.VMEM((1, N_E), jnp.float32)],
    )(z3, cbt)


_SC_CHUNK = 128


def _make_gather():
    info = plsc.get_sparse_core_info()
    nw = info.num_cores * info.num_subcores
    per_w = N_ROWS // nw
    n_chunks = per_w // _SC_CHUNK
    mesh = plsc.VectorSubcoreMesh(core_axis_name="c", subcore_axis_name="s")

    @functools.partial(
        pl.kernel,
        mesh=mesh,
        compiler_params=pltpu.CompilerParams(use_tc_tiling_on_sc=False),
        out_type=jax.ShapeDtypeStruct((N_ROWS, E_DIM), jnp.float32),
        scratch_types=[
            pltpu.VMEM((_SC_CHUNK,), jnp.int32),
            pltpu.VMEM((_SC_CHUNK, E_DIM), jnp.float32),
            pltpu.SemaphoreType.DMA,
        ],
    )
    def gather(table_hbm, idx_hbm, out_hbm, idx_v, rows_v, sem):
        wid = lax.axis_index("s") * info.num_cores + lax.axis_index("c")
        base = wid * per_w
        for k in range(n_chunks):
            off = base + k * _SC_CHUNK
            pltpu.sync_copy(idx_hbm.at[pl.ds(off, _SC_CHUNK)], idx_v)
            pltpu.async_copy(table_hbm.at[idx_v], rows_v, sem).wait()
            pltpu.sync_copy(rows_v, out_hbm.at[pl.ds(off, _SC_CHUNK)])

    return gather


def kernel(z, codebook):
    z3 = z.reshape(z.shape[0], E_DIM, -1)
    cbt = jnp.transpose(codebook)

    idx_blocks, loss_sum = _argmin_call(z3, cbt)
    idx = idx_blocks.reshape(N_ROWS)

    z_q_flat = _make_gather()(codebook, idx)
    z_q = z_q_flat.reshape(z.shape[0], z.shape[2], z.shape[3], E_DIM)
    z_q_out = jnp.transpose(z_q, (0, 3, 1, 2))
    codebook_loss = loss_sum[0, 0] / (N_ROWS * E_DIM)
    idx_map = idx.reshape(z.shape[0], 1, z.shape[2], z.shape[3])
    return (z_q_out, codebook_loss, 0, idx_map)


# speedup vs baseline: 1.2573x; 1.0724x over previous
import functools

import jax
import jax.numpy as jnp
from jax import lax
from jax.experimental import pallas as pl
from jax.experimental.pallas import tpu as pltpu
from jax.experimental.pallas import tpu_sc as plsc

N_E = 8192
E_DIM = 32
N_ROWS = 8192
ROW_BLK = 512
CODE_BLK = 1024
N_ROW_BLKS = N_ROWS // ROW_BLK
N_CODE_BLKS = N_E // CODE_BLK

_MATMUL_PRECISION = lax.Precision.DEFAULT


def _argmin_body(zt_ref, cbt_ref, idx_ref, loss_ref, esq_ref):
    i = pl.program_id(0)
    zt = zt_ref[0]

    @pl.when(i == 0)
    def _():
        cb = cbt_ref[...]
        esq_ref[...] = jnp.sum(cb * cb, axis=0, keepdims=True)

    zsq = jnp.sum(zt * zt, axis=0)[:, None]
    z2 = -2.0 * zt
    LANES = 128
    run_val = jnp.full((ROW_BLK, LANES), jnp.inf, dtype=jnp.float32)
    run_k = jnp.zeros((ROW_BLK, LANES), dtype=jnp.float32)
    for c in range(N_CODE_BLKS):
        sl = slice(c * CODE_BLK, (c + 1) * CODE_BLK)
        cbt = cbt_ref[:, sl]
        m = lax.dot_general(
            z2, cbt, (((0,), (0,)), ((), ())),
            precision=_MATMUL_PRECISION,
            preferred_element_type=jnp.float32)
        for s in range(CODE_BLK // LANES):
            k = c * (CODE_BLK // LANES) + s
            esq = esq_ref[:, k * LANES:(k + 1) * LANES]
            d = (zsq + esq) + m[:, s * LANES:(s + 1) * LANES]
            mask = d < run_val
            run_val = jnp.minimum(d, run_val)
            run_k = jnp.where(mask, jnp.float32(k), run_k)
    gmin = jnp.min(run_val, axis=1, keepdims=True)
    lane = lax.broadcasted_iota(
        jnp.int32, (ROW_BLK, LANES), 1).astype(jnp.float32)
    cand = jnp.where(run_val == gmin, run_k * float(LANES) + lane, 1e9)
    best_col = jnp.min(cand, axis=1)
    idx_ref[0, 0, :] = best_col.astype(jnp.int32)
    part = jnp.sum(gmin)

    @pl.when(i == 0)
    def _():
        loss_ref[0, 0] = part

    @pl.when(i != 0)
    def _():
        loss_ref[0, 0] += part


def _argmin_call(z3, cbt):
    blks_per_batch = 1024 // ROW_BLK
    return pl.pallas_call(
        _argmin_body,
        grid=(N_ROW_BLKS,),
        in_specs=[
            pl.BlockSpec((1, E_DIM, ROW_BLK),
                         lambda i: (i // blks_per_batch, 0, i % blks_per_batch)),
            pl.BlockSpec((E_DIM, N_E), lambda i: (0, 0)),
        ],
        out_specs=[
            pl.BlockSpec((1, 1, ROW_BLK), lambda i: (i, 0, 0)),
            pl.BlockSpec(memory_space=pltpu.SMEM, block_shape=(1, 1),
                         index_map=lambda i: (0, 0)),
        ],
        out_shape=[
            jax.ShapeDtypeStruct((N_ROW_BLKS, 1, ROW_BLK), jnp.int32),
            jax.ShapeDtypeStruct((1, 1), jnp.float32),
        ],
        scratch_shapes=[pltpu.VMEM((1, N_E), jnp.float32)],
    )(z3, cbt)


_SC_CHUNK = 128


def _make_gather():
    info = plsc.get_sparse_core_info()
    nw = info.num_cores * info.num_subcores
    per_w = N_ROWS // nw
    n_chunks = per_w // _SC_CHUNK
    mesh = plsc.VectorSubcoreMesh(core_axis_name="c", subcore_axis_name="s")

    @functools.partial(
        pl.kernel,
        mesh=mesh,
        compiler_params=pltpu.CompilerParams(use_tc_tiling_on_sc=False),
        out_type=jax.ShapeDtypeStruct((N_ROWS, E_DIM), jnp.float32),
        scratch_types=[
            pltpu.VMEM((_SC_CHUNK,), jnp.int32),
            pltpu.VMEM((_SC_CHUNK, E_DIM), jnp.float32),
            pltpu.SemaphoreType.DMA,
        ],
    )
    def gather(table_hbm, idx_hbm, out_hbm, idx_v, rows_v, sem):
        wid = lax.axis_index("s") * info.num_cores + lax.axis_index("c")
        base = wid * per_w
        for k in range(n_chunks):
            off = base + k * _SC_CHUNK
            pltpu.sync_copy(idx_hbm.at[pl.ds(off, _SC_CHUNK)], idx_v)
            pltpu.async_copy(table_hbm.at[idx_v], rows_v, sem).wait()
            pltpu.sync_copy(rows_v, out_hbm.at[pl.ds(off, _SC_CHUNK)])

    return gather


def kernel(z, codebook):
    z3 = z.reshape(z.shape[0], E_DIM, -1)
    cbt = jnp.transpose(codebook)

    idx_blocks, loss_sum = _argmin_call(z3, cbt)
    idx = idx_blocks.reshape(N_ROWS)

    z_q_flat = _make_gather()(codebook, idx)
    z_q = z_q_flat.reshape(z.shape[0], z.shape[2], z.shape[3], E_DIM)
    z_q_out = jnp.transpose(z_q, (0, 3, 1, 2))
    codebook_loss = loss_sum[0, 0] / (N_ROWS * E_DIM)
    idx_map = idx.reshape(z.shape[0], 1, z.shape[2], z.shape[3])
    return (z_q_out, codebook_loss, 0, idx_map)
